# baseline (speedup 1.0000x reference)
import jax
import jax.numpy as jnp
from jax import lax
from jax.experimental import pallas as pl
from jax.experimental.pallas import tpu as pltpu
from jax.experimental.pallas import tpu_sc as plsc

N = 10000
E = 320000
H = 128
G = 64
DW = 16

NC = 2
NS = 16
NW = NC * NS

KE = 128
SB = 8
NSC = 10
NCH = NSC * SB
EP = NW * NCH * KE
NCHR = 79
NR = NCHR * KE
NZ = (NCHR + NS - 1) // NS

import functools


@functools.cache
def _mesh():
    return plsc.VectorSubcoreMesh(core_axis_name="c", subcore_axis_name="s",
                                  num_cores=NC, num_subcores=NS)


def _sc_agg_body(h_hbm, srcb, dstb, z128, agg_out,
                 src_v, dst_v, rows_v, aggS, gsem):
    c = lax.axis_index("c")
    s = lax.axis_index("s")
    wid = c * NS + s

    pltpu.sync_copy(z128, rows_v)
    for k in range(NZ):
        r = s + k * NS

        @pl.when(r < NCHR)
        def _zero():
            pltpu.sync_copy(rows_v, aggS.at[pl.ds(r * KE, KE)])

    plsc.subcore_barrier()

    @pl.loop(0, NSC)
    def _superchunk(i):
        pltpu.sync_copy(srcb.at[wid, pl.ds(i * SB, SB)], src_v)
        pltpu.sync_copy(dstb.at[wid, pl.ds(i * SB, SB)], dst_v)
        for j in range(SB):
            pltpu.async_copy(h_hbm.at[src_v.at[j]], rows_v, gsem).wait()
            pltpu.sync_copy(rows_v, aggS.at[dst_v.at[j]], add=True)

    plsc.subcore_barrier()

    for k in range(NZ):
        r = s + k * NS

        @pl.when(r < NCHR)
        def _drain():
            pltpu.sync_copy(aggS.at[pl.ds(r * KE, KE)], rows_v)
            pltpu.sync_copy(rows_v, agg_out.at[c, pl.ds(r * KE, KE)])


@functools.cache
def _sc_agg():
    return pl.kernel(
        _sc_agg_body,
        out_type=jax.ShapeDtypeStruct((NC, NR, H), jnp.float32),
        mesh=_mesh(),
        scratch_types=[
            pltpu.VMEM((SB, KE), jnp.int32),
            pltpu.VMEM((SB, KE), jnp.int32),
            pltpu.VMEM((KE, H), jnp.float32),
            pltpu.VMEM_SHARED((NR, H), jnp.float32),
            pltpu.SemaphoreType.DMA,
        ],
    )


def _update_body(p_ref, deg_ref, h_ref, wl_ref, b_ref, wr_ref, o_ref):
    agg = p_ref[0, :N, :] + p_ref[1, :N, :]
    d = deg_ref[0, :N, 0:1] + deg_ref[1, :N, 0:1]
    invd = 1.0 / jnp.maximum(d, 1.0)
    m = jnp.dot(agg, wl_ref[:, :], preferred_element_type=jnp.float32) * invd
    r = jnp.dot(h_ref[:, :], wr_ref[:, :], preferred_element_type=jnp.float32)
    o_ref[:, :] = jnp.maximum(m + b_ref[:, :] + r, 0.0)


_tc_update = pl.pallas_call(
    _update_body,
    out_shape=jax.ShapeDtypeStruct((N, H), jnp.float32),
)


def _head_body(h_ref, b_ref, wg_ref, bg_ref, w1_ref, b1_ref, w2_ref, b2_ref,
               o_ref):
    h = h_ref[:, :]
    gate = jnp.dot(h, wg_ref[:, :], preferred_element_type=jnp.float32)
    gate = gate + bg_ref[0, 0]
    ids = lax.broadcasted_iota(jnp.int32, (1, G), 1)
    member = b_ref[:, :] == ids
    masked = jnp.where(member, gate, -1e30)
    gmax = jnp.max(masked, axis=0, keepdims=True)
    egm = jnp.where(member, jnp.exp(masked - gmax), 0.0)
    denom = jnp.sum(egm, axis=0, keepdims=True)
    alpha_m = egm / (denom + 1e-16)
    g = lax.dot_general(alpha_m, h,
                        (((0,), (0,)), ((), ())),
                        preferred_element_type=jnp.float32)
    g1 = jnp.dot(g, w1_ref[:, :], preferred_element_type=jnp.float32)
    g1 = jnp.maximum(g1 + b1_ref[:, :], 0.0)
    o = jnp.dot(g1, w2_ref[:, :], preferred_element_type=jnp.float32)
    o = o + b2_ref[:, :]
    o_ref[:, :] = 1.0 / (1.0 + jnp.exp(-o))


_tc_head = pl.pallas_call(
    _head_body,
    out_shape=jax.ShapeDtypeStruct((G, 1), jnp.float32),
)


def kernel(x, edge_index, batch, W1_l, b1_l, W1_r, W2_l, b2_l, W2_r,
           W3_l, b3_l, W3_r, Wg, bg, Wlin1, blin1, Wlin2, blin2):
    src = edge_index[0]
    dst = edge_index[1]
    pad = EP - E
    srcb = jnp.concatenate(
        [src, jnp.zeros((pad,), jnp.int32)]).reshape(NW, NCH, KE)
    dstb = jnp.concatenate(
        [dst, jnp.full((pad,), N, jnp.int32)]).reshape(NW, NCH, KE)
    z128 = jnp.zeros((KE, H), jnp.float32)
    ones_tab = jnp.ones((8, H), jnp.float32)
    srcz = jnp.zeros((NW, NCH, KE), jnp.int32)

    b1 = b1_l.reshape(1, H)
    b2 = b2_l.reshape(1, H)
    b3 = b3_l.reshape(1, H)

    sc_agg = _sc_agg()
    deg2 = sc_agg(ones_tab, srcz, dstb, z128)
    agg1 = sc_agg(x, srcb, dstb, z128)
    h1 = _tc_update(agg1, deg2, x, W1_l, b1, W1_r)
    agg2 = sc_agg(h1, srcb, dstb, z128)
    h2 = _tc_update(agg2, deg2, h1, W2_l, b2, W2_r)
    agg3 = sc_agg(h2, srcb, dstb, z128)
    h3 = _tc_update(agg3, deg2, h2, W3_l, b3, W3_r)

    out = _tc_head(h3, batch.reshape(N, 1), Wg, bg.reshape(1, 1),
                   Wlin1, blin1.reshape(1, H), Wlin2, blin2.reshape(1, 1))
    return out


# speedup vs baseline: 7.9764x; 7.9764x over previous
import jax
import jax.numpy as jnp
from jax import lax
from jax.experimental import pallas as pl
from jax.experimental.pallas import tpu as pltpu
from jax.experimental.pallas import tpu_sc as plsc

N = 10000
E = 320000
H = 128
G = 64
DW = 16

NC = 2
NS = 16
NW = NC * NS

KE = 128
SB = 8
NSC = 10
NCH = NSC * SB
EP = NW * NCH * KE
NCHR = 79
NR = NCHR * KE
NZ = (NCHR + NS - 1) // NS

import functools


@functools.cache
def _mesh():
    return plsc.VectorSubcoreMesh(core_axis_name="c", subcore_axis_name="s",
                                  num_cores=NC, num_subcores=NS)


def _sc_agg_body(h_hbm, srcb, dstb, z128, agg_out,
                 src_v, dst_v, rows_v, aggS, gsem):
    c = lax.axis_index("c")
    s = lax.axis_index("s")
    wid = c * NS + s

    pltpu.sync_copy(z128, rows_v)
    for k in range(NZ):
        r = s + k * NS

        @pl.when(r < NCHR)
        def _zero():
            pltpu.sync_copy(rows_v, aggS.at[pl.ds(r * KE, KE)])

    plsc.subcore_barrier()

    @pl.loop(0, NSC)
    def _superchunk(i):
        pltpu.sync_copy(srcb.at[wid, pl.ds(i * SB, SB)], src_v)
        pltpu.sync_copy(dstb.at[wid, pl.ds(i * SB, SB)], dst_v)
        for j in range(SB):
            pltpu.async_copy(h_hbm.at[src_v.at[j]], rows_v, gsem).wait()
            pltpu.sync_copy(rows_v, aggS.at[dst_v.at[j]], add=True)

    plsc.subcore_barrier()

    for k in range(NZ):
        r = s + k * NS

        @pl.when(r < NCHR)
        def _drain():
            pltpu.sync_copy(aggS.at[pl.ds(r * KE, KE)], rows_v)
            pltpu.sync_copy(rows_v, agg_out.at[c, pl.ds(r * KE, KE)])


@functools.cache
def _sc_agg():
    return pl.kernel(
        _sc_agg_body,
        out_type=jax.ShapeDtypeStruct((NC, NR, H), jnp.float32),
        mesh=_mesh(),
        scratch_types=[
            pltpu.VMEM((SB, KE), jnp.int32),
            pltpu.VMEM((SB, KE), jnp.int32),
            pltpu.VMEM((KE, H), jnp.float32),
            pltpu.VMEM_SHARED((NR, H), jnp.float32),
            pltpu.SemaphoreType.DMA,
        ],
    )


def _sc_deg_body(dstb, z128, ones128, deg_out, dst_v, rows_v, degS):
    c = lax.axis_index("c")
    s = lax.axis_index("s")
    wid = c * NS + s

    pltpu.sync_copy(z128, rows_v)
    for k in range(NZ):
        r = s + k * NS

        @pl.when(r < NCHR)
        def _zero():
            pltpu.sync_copy(rows_v, degS.at[pl.ds(r * KE, KE)])

    pltpu.sync_copy(ones128, rows_v)
    plsc.subcore_barrier()

    @pl.loop(0, NSC)
    def _superchunk(i):
        pltpu.sync_copy(dstb.at[wid, pl.ds(i * SB, SB)], dst_v)
        for j in range(SB):
            pltpu.sync_copy(rows_v, degS.at[dst_v.at[j]], add=True)

    plsc.subcore_barrier()

    for k in range(NZ):
        r = s + k * NS

        @pl.when(r < NCHR)
        def _drain():
            pltpu.sync_copy(degS.at[pl.ds(r * KE, KE)], rows_v)
            pltpu.sync_copy(rows_v, deg_out.at[c, pl.ds(r * KE, KE)])


@functools.cache
def _sc_deg():
    return pl.kernel(
        _sc_deg_body,
        out_type=jax.ShapeDtypeStruct((NC, NR, H), jnp.float32),
        mesh=_mesh(),
        scratch_types=[
            pltpu.VMEM((SB, KE), jnp.int32),
            pltpu.VMEM((KE, H), jnp.float32),
            pltpu.VMEM_SHARED((NR, H), jnp.float32),
        ],
    )


def _update_body(p_ref, deg_ref, h_ref, wl_ref, b_ref, wr_ref, o_ref):
    agg = p_ref[0, :N, :] + p_ref[1, :N, :]
    d = deg_ref[0, :N, 0:1] + deg_ref[1, :N, 0:1]
    invd = 1.0 / jnp.maximum(d, 1.0)
    m = jnp.dot(agg, wl_ref[:, :], preferred_element_type=jnp.float32) * invd
    r = jnp.dot(h_ref[:, :], wr_ref[:, :], preferred_element_type=jnp.float32)
    o_ref[:, :] = jnp.maximum(m + b_ref[:, :] + r, 0.0)


_tc_update = pl.pallas_call(
    _update_body,
    out_shape=jax.ShapeDtypeStruct((N, H), jnp.float32),
)


def _head_body(h_ref, b_ref, wg_ref, bg_ref, w1_ref, b1_ref, w2_ref, b2_ref,
               o_ref):
    h = h_ref[:, :]
    gate = jnp.dot(h, wg_ref[:, :], preferred_element_type=jnp.float32)
    gate = gate + bg_ref[0, 0]
    ids = lax.broadcasted_iota(jnp.int32, (1, G), 1)
    member = b_ref[:, :] == ids
    masked = jnp.where(member, gate, -1e30)
    gmax = jnp.max(masked, axis=0, keepdims=True)
    egm = jnp.where(member, jnp.exp(masked - gmax), 0.0)
    denom = jnp.sum(egm, axis=0, keepdims=True)
    alpha_m = egm / (denom + 1e-16)
    g = lax.dot_general(alpha_m, h,
                        (((0,), (0,)), ((), ())),
                        preferred_element_type=jnp.float32)
    g1 = jnp.dot(g, w1_ref[:, :], preferred_element_type=jnp.float32)
    g1 = jnp.maximum(g1 + b1_ref[:, :], 0.0)
    o = jnp.dot(g1, w2_ref[:, :], preferred_element_type=jnp.float32)
    o = o + b2_ref[:, :]
    o_ref[:, :] = 1.0 / (1.0 + jnp.exp(-o))


_tc_head = pl.pallas_call(
    _head_body,
    out_shape=jax.ShapeDtypeStruct((G, 1), jnp.float32),
)


def kernel(x, edge_index, batch, W1_l, b1_l, W1_r, W2_l, b2_l, W2_r,
           W3_l, b3_l, W3_r, Wg, bg, Wlin1, blin1, Wlin2, blin2):
    src = edge_index[0]
    dst = edge_index[1]
    pad = EP - E
    srcb = jnp.concatenate(
        [src, jnp.zeros((pad,), jnp.int32)]).reshape(NW, NCH, KE)
    dstb = jnp.concatenate(
        [dst, jnp.full((pad,), N, jnp.int32)]).reshape(NW, NCH, KE)
    z128 = jnp.zeros((KE, H), jnp.float32)
    ones128 = jnp.ones((KE, H), jnp.float32)

    b1 = b1_l.reshape(1, H)
    b2 = b2_l.reshape(1, H)
    b3 = b3_l.reshape(1, H)

    sc_agg = _sc_agg()
    deg2 = _sc_deg()(dstb, z128, ones128)
    agg1 = sc_agg(x, srcb, dstb, z128)
    h1 = _tc_update(agg1, deg2, x, W1_l, b1, W1_r)
    agg2 = sc_agg(h1, srcb, dstb, z128)
    h2 = _tc_update(agg2, deg2, h1, W2_l, b2, W2_r)
    agg3 = sc_agg(h2, srcb, dstb, z128)
    h3 = _tc_update(agg3, deg2, h2, W3_l, b3, W3_r)

    out = _tc_head(h3, batch.reshape(N, 1), Wg, bg.reshape(1, 1),
                   Wlin1, blin1.reshape(1, H), Wlin2, blin2.reshape(1, 1))
    return out
